# baseline (speedup 1.0000x reference)
import functools

import jax
import jax.numpy as jnp
from jax import lax
from jax.experimental import pallas as pl
from jax.experimental.pallas import tpu as pltpu, tpu_sc as plsc

BATCH = 4096
SEQ = 50
EMBED_DIM = 64
N = BATCH * SEQ

NUM_CORES = 2
NUM_SUBCORES = 16
NUM_WORKERS = NUM_CORES * NUM_SUBCORES
PER_WORKER = N // NUM_WORKERS
CHUNK = 128
NUM_CHUNKS = PER_WORKER // CHUNK
NUM_PAIRS = NUM_CHUNKS // 2
LANES = 16


def _emb_sum_kernel(iw_hbm, is_hbm, words_hbm, scores_hbm, out_hbm,
                    idxw_v, idxs_v,
                    wbuf0, wbuf1, sbuf0, sbuf1, obuf0, obuf1,
                    semw0, semw1, sems0, sems1, semo0, semo1):
    wid = lax.axis_index("s") * NUM_CORES + lax.axis_index("c")
    base = wid * PER_WORKER
    wbuf = (wbuf0, wbuf1)
    sbuf = (sbuf0, sbuf1)
    obuf = (obuf0, obuf1)
    semw = (semw0, semw1)
    sems = (sems0, sems1)
    semo = (semo0, semo1)

    pltpu.sync_copy(iw_hbm.at[pl.ds(base, PER_WORKER)], idxw_v)
    pltpu.sync_copy(is_hbm.at[pl.ds(base, PER_WORKER)], idxs_v)

    def gather_into(c, p):
        off = c * CHUNK
        pltpu.async_copy(
            words_hbm.at[idxw_v.at[pl.ds(off, CHUNK)]], wbuf[p], semw[p])
        pltpu.async_copy(
            scores_hbm.at[idxs_v.at[pl.ds(off, CHUNK)]], sbuf[p], sems[p])

    gather_into(0, 0)
    gather_into(1, 1)

    def pair_body(i, carry):
        for p in range(2):
            c = i * 2 + p
            off = c * CHUNK
            pltpu.make_async_copy(
                words_hbm.at[idxw_v.at[pl.ds(off, CHUNK)]],
                wbuf[p], semw[p]).wait()
            pltpu.make_async_copy(
                scores_hbm.at[idxs_v.at[pl.ds(off, CHUNK)]],
                sbuf[p], sems[p]).wait()

            @pl.when(i >= 1)
            def _wait_prev_scatter():
                pltpu.make_async_copy(
                    obuf[p], out_hbm.at[pl.ds(base + off, CHUNK)],
                    semo[p]).wait()

            def add_row(r, carry2):
                for j in range(EMBED_DIM // LANES):
                    sl = pl.ds(j * LANES, LANES)
                    obuf[p][r, sl] = wbuf[p][r, sl] + sbuf[p][r, sl]
                return carry2

            lax.fori_loop(0, CHUNK, add_row, 0, unroll=8)

            pltpu.async_copy(
                obuf[p], out_hbm.at[pl.ds(base + off, CHUNK)], semo[p])

            @pl.when(i < NUM_PAIRS - 1)
            def _prefetch():
                gather_into(c + 2, p)
        return carry

    lax.fori_loop(0, NUM_PAIRS, pair_body, 0)

    for p in range(2):
        off = (NUM_CHUNKS - 2 + p) * CHUNK
        pltpu.make_async_copy(
            obuf[p], out_hbm.at[pl.ds(base + off, CHUNK)], semo[p]).wait()


@jax.jit
def kernel(input_ids, scores_ids, words_table, scores_table):
    iw = input_ids.reshape(-1).astype(jnp.int32)
    isc = scores_ids.reshape(-1).astype(jnp.int32)
    mesh = plsc.VectorSubcoreMesh(core_axis_name="c", subcore_axis_name="s")
    run = functools.partial(
        pl.kernel,
        mesh=mesh,
        compiler_params=pltpu.CompilerParams(use_tc_tiling_on_sc=False),
        out_type=jax.ShapeDtypeStruct((N, EMBED_DIM), jnp.float32),
        scratch_types=[
            pltpu.VMEM((PER_WORKER,), jnp.int32),
            pltpu.VMEM((PER_WORKER,), jnp.int32),
            pltpu.VMEM((CHUNK, EMBED_DIM), jnp.float32),
            pltpu.VMEM((CHUNK, EMBED_DIM), jnp.float32),
            pltpu.VMEM((CHUNK, EMBED_DIM), jnp.float32),
            pltpu.VMEM((CHUNK, EMBED_DIM), jnp.float32),
            pltpu.VMEM((CHUNK, EMBED_DIM), jnp.float32),
            pltpu.VMEM((CHUNK, EMBED_DIM), jnp.float32),
            pltpu.SemaphoreType.DMA,
            pltpu.SemaphoreType.DMA,
            pltpu.SemaphoreType.DMA,
            pltpu.SemaphoreType.DMA,
            pltpu.SemaphoreType.DMA,
            pltpu.SemaphoreType.DMA,
        ],
    )(_emb_sum_kernel)
    out = run(iw, isc, words_table, scores_table)
    return out.reshape(BATCH, SEQ, EMBED_DIM)


# speedup vs baseline: 1.0034x; 1.0034x over previous
import functools

import jax
import jax.numpy as jnp
from jax import lax
from jax.experimental import pallas as pl
from jax.experimental.pallas import tpu as pltpu, tpu_sc as plsc

BATCH = 4096
SEQ = 50
EMBED_DIM = 64
N = BATCH * SEQ

NUM_CORES = 2
NUM_SUBCORES = 16
NUM_WORKERS = NUM_CORES * NUM_SUBCORES
PER_WORKER = N // NUM_WORKERS
CHUNK = 128
NUM_CHUNKS = PER_WORKER // CHUNK
NUM_PAIRS = NUM_CHUNKS // 2
LANES = 16


def _emb_sum_kernel(iw_hbm, is_hbm, words_hbm, scores_hbm, out_hbm,
                    idxw_v, idxs_v,
                    wbuf0, wbuf1, sbuf0, sbuf1, obuf0, obuf1,
                    semw0, semw1, sems0, sems1, semo0, semo1):
    wid = lax.axis_index("s") * NUM_CORES + lax.axis_index("c")
    base = wid * PER_WORKER
    wbuf = (wbuf0, wbuf1)
    sbuf = (sbuf0, sbuf1)
    obuf = (obuf0, obuf1)
    semw = (semw0, semw1)
    sems = (sems0, sems1)
    semo = (semo0, semo1)

    pltpu.sync_copy(iw_hbm.at[pl.ds(base, PER_WORKER)], idxw_v)
    pltpu.sync_copy(is_hbm.at[pl.ds(base, PER_WORKER)], idxs_v)

    def gather_into(c, p):
        off = c * CHUNK
        pltpu.async_copy(
            words_hbm.at[idxw_v.at[pl.ds(off, CHUNK)]], wbuf[p], semw[p])
        pltpu.async_copy(
            scores_hbm.at[idxs_v.at[pl.ds(off, CHUNK)]], sbuf[p], sems[p])

    gather_into(0, 0)
    gather_into(1, 1)

    def pair_body(i, carry):
        for p in range(2):
            c = i * 2 + p
            off = c * CHUNK
            pltpu.make_async_copy(
                words_hbm.at[idxw_v.at[pl.ds(off, CHUNK)]],
                wbuf[p], semw[p]).wait()
            pltpu.make_async_copy(
                scores_hbm.at[idxs_v.at[pl.ds(off, CHUNK)]],
                sbuf[p], sems[p]).wait()

            @pl.when(i >= 1)
            def _wait_prev_scatter():
                pltpu.make_async_copy(
                    obuf[p], out_hbm.at[pl.ds(base + off, CHUNK)],
                    semo[p]).wait()

            @plsc.parallel_loop(0, CHUNK, unroll=4)
            def _add_row(r):
                for j in range(EMBED_DIM // LANES):
                    sl = pl.ds(j * LANES, LANES)
                    obuf[p][r, sl] = wbuf[p][r, sl] + sbuf[p][r, sl]

            pltpu.async_copy(
                obuf[p], out_hbm.at[pl.ds(base + off, CHUNK)], semo[p])

            @pl.when(i < NUM_PAIRS - 1)
            def _prefetch():
                gather_into(c + 2, p)
        return carry

    lax.fori_loop(0, NUM_PAIRS, pair_body, 0)

    for p in range(2):
        off = (NUM_CHUNKS - 2 + p) * CHUNK
        pltpu.make_async_copy(
            obuf[p], out_hbm.at[pl.ds(base + off, CHUNK)], semo[p]).wait()


@jax.jit
def kernel(input_ids, scores_ids, words_table, scores_table):
    iw = input_ids.reshape(-1).astype(jnp.int32)
    isc = scores_ids.reshape(-1).astype(jnp.int32)
    mesh = plsc.VectorSubcoreMesh(core_axis_name="c", subcore_axis_name="s")
    run = functools.partial(
        pl.kernel,
        mesh=mesh,
        compiler_params=pltpu.CompilerParams(use_tc_tiling_on_sc=False),
        out_type=jax.ShapeDtypeStruct((N, EMBED_DIM), jnp.float32),
        scratch_types=[
            pltpu.VMEM((PER_WORKER,), jnp.int32),
            pltpu.VMEM((PER_WORKER,), jnp.int32),
            pltpu.VMEM((CHUNK, EMBED_DIM), jnp.float32),
            pltpu.VMEM((CHUNK, EMBED_DIM), jnp.float32),
            pltpu.VMEM((CHUNK, EMBED_DIM), jnp.float32),
            pltpu.VMEM((CHUNK, EMBED_DIM), jnp.float32),
            pltpu.VMEM((CHUNK, EMBED_DIM), jnp.float32),
            pltpu.VMEM((CHUNK, EMBED_DIM), jnp.float32),
            pltpu.SemaphoreType.DMA,
            pltpu.SemaphoreType.DMA,
            pltpu.SemaphoreType.DMA,
            pltpu.SemaphoreType.DMA,
            pltpu.SemaphoreType.DMA,
            pltpu.SemaphoreType.DMA,
        ],
    )(_emb_sum_kernel)
    out = run(iw, isc, words_table, scores_table)
    return out.reshape(BATCH, SEQ, EMBED_DIM)


# speedup vs baseline: 3.3639x; 3.3524x over previous
import functools

import jax
import jax.numpy as jnp
from jax import lax
from jax.experimental import pallas as pl
from jax.experimental.pallas import tpu as pltpu, tpu_sc as plsc

BATCH = 4096
SEQ = 50
EMBED_DIM = 64
N = BATCH * SEQ

NUM_CORES = 2
NUM_SUBCORES = 16
NUM_WORKERS = NUM_CORES * NUM_SUBCORES
PER_WORKER = N // NUM_WORKERS
CHUNK = 128
NUM_CHUNKS = PER_WORKER // CHUNK
NUM_PAIRS = NUM_CHUNKS // 2
LANES = 16
NUM_BINS = 12


def _emb_sum_kernel(iw_hbm, is_hbm, words_hbm, scores_hbm, out_hbm,
                    idxw_v, idxs_v, stab,
                    wbuf0, wbuf1, obuf0, obuf1,
                    semw0, semw1, semo0, semo1):
    wid = lax.axis_index("s") * NUM_CORES + lax.axis_index("c")
    base = wid * PER_WORKER
    wbuf = (wbuf0, wbuf1)
    obuf = (obuf0, obuf1)
    semw = (semw0, semw1)
    semo = (semo0, semo1)

    pltpu.sync_copy(iw_hbm.at[pl.ds(base, PER_WORKER)], idxw_v)
    pltpu.sync_copy(is_hbm.at[pl.ds(base, PER_WORKER)], idxs_v)
    pltpu.sync_copy(scores_hbm, stab)

    def gather_into(c, p):
        off = c * CHUNK
        pltpu.async_copy(
            words_hbm.at[idxw_v.at[pl.ds(off, CHUNK)]], wbuf[p], semw[p])

    gather_into(0, 0)
    gather_into(1, 1)

    def pair_body(i, carry):
        for p in range(2):
            c = i * 2 + p
            off = c * CHUNK
            pltpu.make_async_copy(
                words_hbm.at[idxw_v.at[pl.ds(off, CHUNK)]],
                wbuf[p], semw[p]).wait()

            @pl.when(i >= 1)
            def _wait_prev_scatter():
                pltpu.make_async_copy(
                    obuf[p], out_hbm.at[pl.ds(base + off, CHUNK)],
                    semo[p]).wait()

            @plsc.parallel_loop(0, CHUNK, step=LANES)
            def _add_group(g):
                sidv = idxs_v[pl.ds(off + g, LANES)]
                for k in range(LANES):
                    sid = sidv[k]
                    for j in range(EMBED_DIM // LANES):
                        sl = pl.ds(j * LANES, LANES)
                        obuf[p][g + k, sl] = wbuf[p][g + k, sl] + stab[sid, sl]

            pltpu.async_copy(
                obuf[p], out_hbm.at[pl.ds(base + off, CHUNK)], semo[p])

            @pl.when(i < NUM_PAIRS - 1)
            def _prefetch():
                gather_into(c + 2, p)
        return carry

    lax.fori_loop(0, NUM_PAIRS, pair_body, 0)

    for p in range(2):
        off = (NUM_CHUNKS - 2 + p) * CHUNK
        pltpu.make_async_copy(
            obuf[p], out_hbm.at[pl.ds(base + off, CHUNK)], semo[p]).wait()


@jax.jit
def kernel(input_ids, scores_ids, words_table, scores_table):
    iw = input_ids.reshape(-1).astype(jnp.int32)
    isc = scores_ids.reshape(-1).astype(jnp.int32)
    mesh = plsc.VectorSubcoreMesh(core_axis_name="c", subcore_axis_name="s")
    run = functools.partial(
        pl.kernel,
        mesh=mesh,
        compiler_params=pltpu.CompilerParams(use_tc_tiling_on_sc=False),
        out_type=jax.ShapeDtypeStruct((N, EMBED_DIM), jnp.float32),
        scratch_types=[
            pltpu.VMEM((PER_WORKER,), jnp.int32),
            pltpu.VMEM((PER_WORKER,), jnp.int32),
            pltpu.VMEM((NUM_BINS, EMBED_DIM), jnp.float32),
            pltpu.VMEM((CHUNK, EMBED_DIM), jnp.float32),
            pltpu.VMEM((CHUNK, EMBED_DIM), jnp.float32),
            pltpu.VMEM((CHUNK, EMBED_DIM), jnp.float32),
            pltpu.VMEM((CHUNK, EMBED_DIM), jnp.float32),
            pltpu.SemaphoreType.DMA,
            pltpu.SemaphoreType.DMA,
            pltpu.SemaphoreType.DMA,
            pltpu.SemaphoreType.DMA,
        ],
    )(_emb_sum_kernel)
    out = run(iw, isc, words_table, scores_table)
    return out.reshape(BATCH, SEQ, EMBED_DIM)
